# baseline (speedup 1.0000x reference)
import functools

import jax
import jax.numpy as jnp
from jax import lax
from jax.experimental import pallas as pl
from jax.experimental.pallas import tpu as pltpu
from jax.experimental.pallas import tpu_sc as plsc

NC, NS = 2, 16
NW = NC * NS
B = 16384
CTXW = 20
D = 64
LANES = 16
KD = D // LANES
BPW = B // NW
CB = 32
NCHUNK = BPW // CB
ROWS = CB * CTXW
IDXW = 128
NG = ROWS // IDXW
TG = BPW // IDXW

_mesh = plsc.VectorSubcoreMesh(core_axis_name="c", subcore_axis_name="s")

_GATHER_DNUMS = lax.GatherDimensionNumbers(
    offset_dims=(), collapsed_slice_dims=(0,), start_index_map=(0,))


def _dyn_gather(v, idx):
    return lax.gather(v, idx[:, None], _GATHER_DNUMS, slice_sizes=(1,),
                      mode=lax.GatherScatterMode.PROMISE_IN_BOUNDS)


@functools.partial(
    pl.kernel,
    mesh=_mesh,
    compiler_params=pltpu.CompilerParams(use_tc_tiling_on_sc=False),
    out_type=jax.ShapeDtypeStruct((B,), jnp.float32),
    scratch_types=[
        pltpu.VMEM((NCHUNK * NG, IDXW), jnp.int32),
        pltpu.VMEM((2 * TG, IDXW), jnp.int32),
        pltpu.VMEM((ROWS, D), jnp.float32),
        pltpu.VMEM((BPW, D), jnp.float32),
        pltpu.VMEM((BPW,), jnp.float32),
        pltpu.SemaphoreType.DMA,
        pltpu.SemaphoreType.DMA,
    ],
)
def _cbow_sc(x2, y2, wx, wy, out, cidx, tidx, ctx_v, tgt_v, out_v, sem, tsem):
    wid = lax.axis_index("s") * NC + lax.axis_index("c")

    pltpu.sync_copy(x2.at[pl.ds(wid * (NCHUNK * NG), NCHUNK * NG)], cidx)
    pltpu.sync_copy(y2.at[pl.ds((wid // 2) * (2 * TG), 2 * TG)], tidx)
    troff = (wid % 2) * TG
    tcopies = [
        pltpu.async_copy(wy.at[tidx.at[troff + j]],
                         tgt_v.at[pl.ds(j * IDXW, IDXW)], tsem)
        for j in range(TG)
    ]
    for cpy in tcopies:
        cpy.wait()

    def chunk_body(g, _):
        copies = [
            pltpu.async_copy(wx.at[cidx.at[g * NG + j]],
                             ctx_v.at[pl.ds(j * IDXW, IDXW)], sem)
            for j in range(NG)
        ]
        for cpy in copies:
            cpy.wait()

        lane = lax.broadcasted_iota(jnp.int32, (LANES,), 0)

        def group_body(q, _):
            def batch_body(i, res):
                b = q * LANES + i
                r0 = b * CTXW
                accs = [ctx_v[r0, pl.ds(k * LANES, LANES)] for k in range(KD)]
                for c in range(1, CTXW):
                    for k in range(KD):
                        accs[k] = accs[k] + ctx_v[r0 + c, pl.ds(k * LANES, LANES)]
                tb = g * CB + b
                s = None
                for k in range(KD):
                    term = accs[k] * tgt_v[tb, pl.ds(k * LANES, LANES)]
                    s = term if s is None else s + term
                s = s * (1.0 / CTXW)
                for off in (8, 4, 2, 1):
                    s = s + _dyn_gather(s, lane ^ off)
                return jnp.where(lane == i, s, res)

            res = lax.fori_loop(0, LANES, batch_body,
                                jnp.zeros((LANES,), jnp.float32))
            out_v[pl.ds((g * (CB // LANES) + q) * LANES, LANES)] = res
            return 0

        lax.fori_loop(0, CB // LANES, group_body, 0)
        return 0

    lax.fori_loop(0, NCHUNK, chunk_body, 0)

    pltpu.sync_copy(out_v, out.at[pl.ds(wid * BPW, BPW)])


def kernel(X, y, Wx, Wy):
    x2 = X.astype(jnp.int32).reshape(-1, IDXW)
    y2 = y.astype(jnp.int32).reshape(-1, IDXW)
    return _cbow_sc(x2, y2, Wx, Wy)


# speedup vs baseline: 1.0236x; 1.0236x over previous
import functools

import jax
import jax.numpy as jnp
from jax import lax
from jax.experimental import pallas as pl
from jax.experimental.pallas import tpu as pltpu
from jax.experimental.pallas import tpu_sc as plsc

NC, NS = 2, 16
NW = NC * NS
B = 16384
CTXW = 20
D = 64
LANES = 16
KD = D // LANES
BPW = B // NW
CB = 32
NCHUNK = BPW // CB
ROWS = CB * CTXW
IDXW = 128
NG = ROWS // IDXW
TG = BPW // IDXW

_mesh = plsc.VectorSubcoreMesh(core_axis_name="c", subcore_axis_name="s")

_GATHER_DNUMS = lax.GatherDimensionNumbers(
    offset_dims=(), collapsed_slice_dims=(0,), start_index_map=(0,))


def _dyn_gather(v, idx):
    return lax.gather(v, idx[:, None], _GATHER_DNUMS, slice_sizes=(1,),
                      mode=lax.GatherScatterMode.PROMISE_IN_BOUNDS)


@functools.partial(
    pl.kernel,
    mesh=_mesh,
    compiler_params=pltpu.CompilerParams(use_tc_tiling_on_sc=False),
    out_type=jax.ShapeDtypeStruct((B,), jnp.float32),
    scratch_types=[
        pltpu.VMEM((NCHUNK * NG, IDXW), jnp.int32),
        pltpu.VMEM((2 * TG, IDXW), jnp.int32),
        pltpu.VMEM((2, ROWS, D), jnp.float32),
        pltpu.VMEM((BPW, D), jnp.float32),
        pltpu.VMEM((BPW,), jnp.float32),
        pltpu.SemaphoreType.DMA,
        pltpu.SemaphoreType.DMA,
    ],
)
def _cbow_sc(x2, y2, wx, wy, out, cidx, tidx, ctx_v, tgt_v, out_v, sem, tsem):
    wid = lax.axis_index("s") * NC + lax.axis_index("c")

    pltpu.sync_copy(x2.at[pl.ds(wid * (NCHUNK * NG), NCHUNK * NG)], cidx)
    pltpu.sync_copy(y2.at[pl.ds((wid // 2) * (2 * TG), 2 * TG)], tidx)
    troff = (wid % 2) * TG
    for j in range(TG):
        pltpu.make_async_copy(wy.at[tidx.at[troff + j]],
                              tgt_v.at[pl.ds(j * IDXW, IDXW)], tsem).start()
    pltpu.make_async_copy(wy.at[pl.ds(0, BPW)], tgt_v, tsem).wait()

    lane = lax.broadcasted_iota(jnp.int32, (LANES,), 0)

    def fire_chunk(g, buf):
        for j in range(NG):
            pltpu.make_async_copy(
                wx.at[cidx.at[g * NG + j]],
                ctx_v.at[buf, pl.ds(j * IDXW, IDXW)], sem).start()

    def wait_chunk(buf):
        pltpu.make_async_copy(wx.at[pl.ds(0, ROWS)], ctx_v.at[buf], sem).wait()

    def compute_chunk(g, buf):
        cbuf = ctx_v.at[buf]

        def group_body(q, _):
            def batch_body(i, res):
                b = q * LANES + i
                r0 = b * CTXW
                accs = [cbuf[r0, pl.ds(k * LANES, LANES)] for k in range(KD)]
                for c in range(1, CTXW):
                    for k in range(KD):
                        accs[k] = accs[k] + cbuf[r0 + c, pl.ds(k * LANES, LANES)]
                tb = g * CB + b
                s = None
                for k in range(KD):
                    term = accs[k] * tgt_v[tb, pl.ds(k * LANES, LANES)]
                    s = term if s is None else s + term
                s = s * (1.0 / CTXW)
                for off in (8, 4, 2, 1):
                    s = s + _dyn_gather(s, lane ^ off)
                return jnp.where(lane == i, s, res)

            res = lax.fori_loop(0, LANES, batch_body,
                                jnp.zeros((LANES,), jnp.float32))
            out_v[pl.ds((g * (CB // LANES) + q) * LANES, LANES)] = res
            return 0

        lax.fori_loop(0, CB // LANES, group_body, 0)

    fire_chunk(0, 0)
    for g in range(NCHUNK):
        buf = g % 2
        if g + 1 < NCHUNK:
            fire_chunk(g + 1, 1 - buf)
        wait_chunk(buf)
        compute_chunk(g, buf)

    pltpu.sync_copy(out_v, out.at[pl.ds(wid * BPW, BPW)])


def kernel(X, y, Wx, Wy):
    x2 = X.astype(jnp.int32).reshape(-1, IDXW)
    y2 = y.astype(jnp.int32).reshape(-1, IDXW)
    return _cbow_sc(x2, y2, Wx, Wy)


# speedup vs baseline: 1.5953x; 1.5586x over previous
import functools

import jax
import jax.numpy as jnp
from jax import lax
from jax.experimental import pallas as pl
from jax.experimental.pallas import tpu as pltpu
from jax.experimental.pallas import tpu_sc as plsc

NC, NS = 2, 16
NW = NC * NS
B = 16384
CTXW = 20
D = 64
LANES = 16
KD = D // LANES
BPW = B // NW
CB = 32
NCHUNK = BPW // CB
ROWS = CB * CTXW
IDXW = 128
NG = ROWS // IDXW
TG = BPW // IDXW

_mesh = plsc.VectorSubcoreMesh(core_axis_name="c", subcore_axis_name="s")

VOCAB = 1000000
HALF = 512000
TBN = 2048
TNB = HALF // TBN
_VLAST = (VOCAB - 1) // TBN


def _tr_body(a_ref, b_ref, o_ref):
    o_ref[:, 0:D] = jnp.transpose(a_ref[...])
    o_ref[:, D:2 * D] = jnp.transpose(b_ref[...])


def _tc_transpose(wt):
    return pl.pallas_call(
        _tr_body,
        grid=(TNB,),
        in_specs=[
            pl.BlockSpec((D, TBN), lambda i: (0, i)),
            pl.BlockSpec((D, TBN), lambda i: (0, jnp.minimum(i + TNB, _VLAST))),
        ],
        out_specs=pl.BlockSpec((TBN, 2 * D), lambda i: (i, 0)),
        out_shape=jax.ShapeDtypeStruct((HALF, 2 * D), jnp.float32),
    )(wt, wt)


def _remap(v):
    return jnp.where(v >= HALF, v * 2 - (2 * HALF - 1), v * 2)


_GATHER_DNUMS = lax.GatherDimensionNumbers(
    offset_dims=(), collapsed_slice_dims=(0,), start_index_map=(0,))


def _dyn_gather(v, idx):
    return lax.gather(v, idx[:, None], _GATHER_DNUMS, slice_sizes=(1,),
                      mode=lax.GatherScatterMode.PROMISE_IN_BOUNDS)


@functools.partial(
    pl.kernel,
    mesh=_mesh,
    compiler_params=pltpu.CompilerParams(use_tc_tiling_on_sc=False),
    out_type=jax.ShapeDtypeStruct((B,), jnp.float32),
    scratch_types=[
        pltpu.VMEM((NCHUNK * NG, IDXW), jnp.int32),
        pltpu.VMEM((2 * TG, IDXW), jnp.int32),
        pltpu.VMEM((2, ROWS, D), jnp.float32),
        pltpu.VMEM((BPW, D), jnp.float32),
        pltpu.VMEM((BPW,), jnp.float32),
        pltpu.SemaphoreType.DMA,
        pltpu.SemaphoreType.DMA,
    ],
)
def _cbow_sc(x2, y2, wx, wy, out, cidx, tidx, ctx_v, tgt_v, out_v, sem, tsem):
    wid = lax.axis_index("s") * NC + lax.axis_index("c")

    pltpu.sync_copy(x2.at[pl.ds(wid * (NCHUNK * NG), NCHUNK * NG)], cidx)
    pltpu.sync_copy(y2.at[pl.ds((wid // 2) * (2 * TG), 2 * TG)], tidx)
    troff = (wid % 2) * TG

    def remap_row(r, _):
        for k in range(IDXW // LANES):
            sl = pl.ds(k * LANES, LANES)
            cidx[r, sl] = _remap(cidx[r, sl])
        return 0

    lax.fori_loop(0, NCHUNK * NG, remap_row, 0)
    for j in range(TG):
        for k in range(IDXW // LANES):
            sl = pl.ds(k * LANES, LANES)
            tidx[troff + j, sl] = _remap(tidx[troff + j, sl])

    for j in range(TG):
        pltpu.make_async_copy(wy.at[tidx.at[troff + j]],
                              tgt_v.at[pl.ds(j * IDXW, IDXW)], tsem).start()
    pltpu.make_async_copy(wy.at[pl.ds(0, BPW)], tgt_v, tsem).wait()

    lane = lax.broadcasted_iota(jnp.int32, (LANES,), 0)

    def fire_chunk(g, buf):
        for j in range(NG):
            pltpu.make_async_copy(
                wx.at[cidx.at[g * NG + j]],
                ctx_v.at[buf, pl.ds(j * IDXW, IDXW)], sem).start()

    def wait_chunk(buf):
        pltpu.make_async_copy(wx.at[pl.ds(0, ROWS)], ctx_v.at[buf], sem).wait()

    def compute_chunk(g, buf):
        cbuf = ctx_v.at[buf]

        def group_body(q, _):
            def batch_body(i, res):
                b = q * LANES + i
                r0 = b * CTXW
                accs = [cbuf[r0, pl.ds(k * LANES, LANES)] for k in range(KD)]
                for c in range(1, CTXW):
                    for k in range(KD):
                        accs[k] = accs[k] + cbuf[r0 + c, pl.ds(k * LANES, LANES)]
                tb = g * CB + b
                s = None
                for k in range(KD):
                    term = accs[k] * tgt_v[tb, pl.ds(k * LANES, LANES)]
                    s = term if s is None else s + term
                s = s * (1.0 / CTXW)
                for off in (8, 4, 2, 1):
                    s = s + _dyn_gather(s, lane ^ off)
                return jnp.where(lane == i, s, res)

            res = lax.fori_loop(0, LANES, batch_body,
                                jnp.zeros((LANES,), jnp.float32))
            out_v[pl.ds((g * (CB // LANES) + q) * LANES, LANES)] = res
            return 0

        lax.fori_loop(0, CB // LANES, group_body, 0)

    fire_chunk(0, 0)
    for g in range(NCHUNK):
        buf = g % 2
        if g + 1 < NCHUNK:
            fire_chunk(g + 1, 1 - buf)
        wait_chunk(buf)
        compute_chunk(g, buf)

    pltpu.sync_copy(out_v, out.at[pl.ds(wid * BPW, BPW)])


def kernel(X, y, Wx, Wy):
    x2 = X.astype(jnp.int32).reshape(-1, IDXW)
    y2 = y.astype(jnp.int32).reshape(-1, IDXW)
    wxt = _tc_transpose(Wx.T).reshape(2 * HALF, D)
    wyt = _tc_transpose(Wy.T).reshape(2 * HALF, D)
    return _cbow_sc(x2, y2, wxt, wyt)


# speedup vs baseline: 2.3366x; 1.4647x over previous
import functools

import jax
import jax.numpy as jnp
from jax import lax
from jax.experimental import pallas as pl
from jax.experimental.pallas import tpu as pltpu
from jax.experimental.pallas import tpu_sc as plsc

NC, NS = 2, 16
NW = NC * NS
B = 16384
CTXW = 20
D = 64
LANES = 16
KD = D // LANES
BPW = B // NW
CB = 32
NCHUNK = BPW // CB
ROWS = CB * CTXW
IDXW = 128
NG = ROWS // IDXW
TG = BPW // IDXW

_mesh = plsc.VectorSubcoreMesh(core_axis_name="c", subcore_axis_name="s")

VOCAB = 1000000
HALF = 512000
TBN = 4096
TNB = HALF // TBN
_VLAST = (VOCAB - 1) // TBN


def _tr_body(ax_ref, bx_ref, ay_ref, by_ref, ox_ref, oy_ref):
    ox_ref[...] = jnp.concatenate(
        [jnp.transpose(ax_ref[...]), jnp.transpose(bx_ref[...])], axis=1)
    oy_ref[...] = jnp.concatenate(
        [jnp.transpose(ay_ref[...]), jnp.transpose(by_ref[...])], axis=1)


def _tc_transpose(wxt, wyt):
    spec1 = pl.BlockSpec((D, TBN), lambda i: (0, i))
    spec2 = pl.BlockSpec((D, TBN), lambda i: (0, jnp.minimum(i + TNB, _VLAST)))
    ospec = pl.BlockSpec((TBN, 2 * D), lambda i: (i, 0))
    oshape = jax.ShapeDtypeStruct((HALF, 2 * D), jnp.float32)
    return pl.pallas_call(
        _tr_body,
        grid=(TNB,),
        in_specs=[spec1, spec2, spec1, spec2],
        out_specs=[ospec, ospec],
        out_shape=[oshape, oshape],
    )(wxt, wxt, wyt, wyt)


def _remap(v):
    return jnp.where(v >= HALF, v * 2 - (2 * HALF - 1), v * 2)


_GATHER_DNUMS = lax.GatherDimensionNumbers(
    offset_dims=(), collapsed_slice_dims=(0,), start_index_map=(0,))


def _dyn_gather(v, idx):
    return lax.gather(v, idx[:, None], _GATHER_DNUMS, slice_sizes=(1,),
                      mode=lax.GatherScatterMode.PROMISE_IN_BOUNDS)


@functools.partial(
    pl.kernel,
    mesh=_mesh,
    compiler_params=pltpu.CompilerParams(use_tc_tiling_on_sc=False),
    out_type=jax.ShapeDtypeStruct((B,), jnp.float32),
    scratch_types=[
        pltpu.VMEM((NCHUNK * NG, IDXW), jnp.int32),
        pltpu.VMEM((2 * TG, IDXW), jnp.int32),
        pltpu.VMEM((2, ROWS, D), jnp.float32),
        pltpu.VMEM((BPW, D), jnp.float32),
        pltpu.VMEM((BPW,), jnp.float32),
        pltpu.SemaphoreType.DMA,
        pltpu.SemaphoreType.DMA,
    ],
)
def _cbow_sc(x2, y2, wx, wy, out, cidx, tidx, ctx_v, tgt_v, out_v, sem, tsem):
    wid = lax.axis_index("s") * NC + lax.axis_index("c")

    pltpu.sync_copy(x2.at[pl.ds(wid * (NCHUNK * NG), NCHUNK * NG)], cidx)
    pltpu.sync_copy(y2.at[pl.ds((wid // 2) * (2 * TG), 2 * TG)], tidx)
    troff = (wid % 2) * TG

    def remap_row(r, _):
        for k in range(IDXW // LANES):
            sl = pl.ds(k * LANES, LANES)
            cidx[r, sl] = _remap(cidx[r, sl])
        return 0

    lax.fori_loop(0, NCHUNK * NG, remap_row, 0)
    for j in range(TG):
        for k in range(IDXW // LANES):
            sl = pl.ds(k * LANES, LANES)
            tidx[troff + j, sl] = _remap(tidx[troff + j, sl])

    for j in range(TG):
        pltpu.make_async_copy(wy.at[tidx.at[troff + j]],
                              tgt_v.at[pl.ds(j * IDXW, IDXW)], tsem).start()
    pltpu.make_async_copy(wy.at[pl.ds(0, BPW)], tgt_v, tsem).wait()

    lane = lax.broadcasted_iota(jnp.int32, (LANES,), 0)

    def fire_chunk(g, buf):
        for j in range(NG):
            pltpu.make_async_copy(
                wx.at[cidx.at[g * NG + j]],
                ctx_v.at[buf, pl.ds(j * IDXW, IDXW)], sem).start()

    def wait_chunk(buf):
        pltpu.make_async_copy(wx.at[pl.ds(0, ROWS)], ctx_v.at[buf], sem).wait()

    def compute_chunk(g, buf):
        cbuf = ctx_v.at[buf]

        def group_body(q, _):
            def batch_body(i, res):
                b = q * LANES + i
                r0 = b * CTXW
                accs = [cbuf[r0, pl.ds(k * LANES, LANES)] for k in range(KD)]
                for c in range(1, CTXW):
                    for k in range(KD):
                        accs[k] = accs[k] + cbuf[r0 + c, pl.ds(k * LANES, LANES)]
                tb = g * CB + b
                s = None
                for k in range(KD):
                    term = accs[k] * tgt_v[tb, pl.ds(k * LANES, LANES)]
                    s = term if s is None else s + term
                s = s * (1.0 / CTXW)
                for off in (8, 4, 2, 1):
                    s = s + _dyn_gather(s, lane ^ off)
                return jnp.where(lane == i, s, res)

            res = lax.fori_loop(0, LANES, batch_body,
                                jnp.zeros((LANES,), jnp.float32))
            out_v[pl.ds((g * (CB // LANES) + q) * LANES, LANES)] = res
            return 0

        lax.fori_loop(0, CB // LANES, group_body, 0)

    fire_chunk(0, 0)
    for g in range(NCHUNK):
        buf = g % 2
        if g + 1 < NCHUNK:
            fire_chunk(g + 1, 1 - buf)
        wait_chunk(buf)
        compute_chunk(g, buf)

    pltpu.sync_copy(out_v, out.at[pl.ds(wid * BPW, BPW)])


def kernel(X, y, Wx, Wy):
    x2 = X.astype(jnp.int32).reshape(-1, IDXW)
    y2 = y.astype(jnp.int32).reshape(-1, IDXW)
    wxp, wyp = _tc_transpose(Wx.T, Wy.T)
    wxt = wxp.reshape(2 * HALF, D)
    wyt = wyp.reshape(2 * HALF, D)
    return _cbow_sc(x2, y2, wxt, wyt)


# speedup vs baseline: 2.3707x; 1.0146x over previous
import functools

import jax
import jax.numpy as jnp
from jax import lax
from jax.experimental import pallas as pl
from jax.experimental.pallas import tpu as pltpu
from jax.experimental.pallas import tpu_sc as plsc

NC, NS = 2, 16
NW = NC * NS
B = 16384
CTXW = 20
D = 64
LANES = 16
KD = D // LANES
BPW = B // NW
CB = 32
NCHUNK = BPW // CB
ROWS = CB * CTXW
IDXW = 128
NG = ROWS // IDXW
TG = BPW // IDXW

_mesh = plsc.VectorSubcoreMesh(core_axis_name="c", subcore_axis_name="s")

VOCAB = 1000000
HALF = 512000
TBN = 6400
TNB = HALF // TBN
_VLAST = (VOCAB - 1) // TBN


_TDN = (((0,), (0,)), ((), ()))


def _mt(a, eye):
    return lax.dot_general(a, eye, _TDN, preferred_element_type=jnp.float32)


def _tr_body(ax_ref, bx_ref, ay_ref, by_ref, ox_ref, oy_ref):
    eye = jnp.eye(D, dtype=jnp.float32)
    ox_ref[:, 0:D] = _mt(ax_ref[...], eye)
    ox_ref[:, D:2 * D] = _mt(bx_ref[...], eye)
    oy_ref[:, 0:D] = _mt(ay_ref[...], eye)
    oy_ref[:, D:2 * D] = _mt(by_ref[...], eye)


def _tc_transpose(wxt, wyt):
    spec1 = pl.BlockSpec((D, TBN), lambda i: (0, i))
    spec2 = pl.BlockSpec((D, TBN), lambda i: (0, jnp.minimum(i + TNB, _VLAST)))
    ospec = pl.BlockSpec((TBN, 2 * D), lambda i: (i, 0))
    oshape = jax.ShapeDtypeStruct((HALF, 2 * D), jnp.float32)
    return pl.pallas_call(
        _tr_body,
        grid=(TNB,),
        in_specs=[spec1, spec2, spec1, spec2],
        out_specs=[ospec, ospec],
        out_shape=[oshape, oshape],
    )(wxt, wxt, wyt, wyt)


def _remap(v):
    return jnp.where(v >= HALF, v * 2 - (2 * HALF - 1), v * 2)


_GATHER_DNUMS = lax.GatherDimensionNumbers(
    offset_dims=(), collapsed_slice_dims=(0,), start_index_map=(0,))


def _dyn_gather(v, idx):
    return lax.gather(v, idx[:, None], _GATHER_DNUMS, slice_sizes=(1,),
                      mode=lax.GatherScatterMode.PROMISE_IN_BOUNDS)


@functools.partial(
    pl.kernel,
    mesh=_mesh,
    compiler_params=pltpu.CompilerParams(use_tc_tiling_on_sc=False),
    out_type=jax.ShapeDtypeStruct((B,), jnp.float32),
    scratch_types=[
        pltpu.VMEM((NCHUNK * NG, IDXW), jnp.int32),
        pltpu.VMEM((2 * TG, IDXW), jnp.int32),
        pltpu.VMEM((2, ROWS, D), jnp.float32),
        pltpu.VMEM((BPW, D), jnp.float32),
        pltpu.VMEM((BPW,), jnp.float32),
        pltpu.SemaphoreType.DMA,
        pltpu.SemaphoreType.DMA,
    ],
)
def _cbow_sc(x2, y2, wx, wy, out, cidx, tidx, ctx_v, tgt_v, out_v, sem, tsem):
    wid = lax.axis_index("s") * NC + lax.axis_index("c")

    pltpu.sync_copy(x2.at[pl.ds(wid * (NCHUNK * NG), NCHUNK * NG)], cidx)
    pltpu.sync_copy(y2.at[pl.ds((wid // 2) * (2 * TG), 2 * TG)], tidx)
    troff = (wid % 2) * TG

    def remap_row(r, _):
        for k in range(IDXW // LANES):
            sl = pl.ds(k * LANES, LANES)
            cidx[r, sl] = _remap(cidx[r, sl])
        return 0

    lax.fori_loop(0, NCHUNK * NG, remap_row, 0)
    for j in range(TG):
        for k in range(IDXW // LANES):
            sl = pl.ds(k * LANES, LANES)
            tidx[troff + j, sl] = _remap(tidx[troff + j, sl])

    for j in range(TG):
        pltpu.make_async_copy(wy.at[tidx.at[troff + j]],
                              tgt_v.at[pl.ds(j * IDXW, IDXW)], tsem).start()
    pltpu.make_async_copy(wy.at[pl.ds(0, BPW)], tgt_v, tsem).wait()

    lane = lax.broadcasted_iota(jnp.int32, (LANES,), 0)

    def fire_chunk(g, buf):
        for j in range(NG):
            pltpu.make_async_copy(
                wx.at[cidx.at[g * NG + j]],
                ctx_v.at[buf, pl.ds(j * IDXW, IDXW)], sem).start()

    def wait_chunk(buf):
        pltpu.make_async_copy(wx.at[pl.ds(0, ROWS)], ctx_v.at[buf], sem).wait()

    def compute_chunk(g, buf):
        cbuf = ctx_v.at[buf]

        def group_body(q, _):
            def batch_body(i, res):
                b = q * LANES + i
                r0 = b * CTXW
                accs = [cbuf[r0, pl.ds(k * LANES, LANES)] for k in range(KD)]
                for c in range(1, CTXW):
                    for k in range(KD):
                        accs[k] = accs[k] + cbuf[r0 + c, pl.ds(k * LANES, LANES)]
                tb = g * CB + b
                s = None
                for k in range(KD):
                    term = accs[k] * tgt_v[tb, pl.ds(k * LANES, LANES)]
                    s = term if s is None else s + term
                s = s * (1.0 / CTXW)
                for off in (8, 4, 2, 1):
                    s = s + _dyn_gather(s, lane ^ off)
                return jnp.where(lane == i, s, res)

            res = lax.fori_loop(0, LANES, batch_body,
                                jnp.zeros((LANES,), jnp.float32))
            out_v[pl.ds((g * (CB // LANES) + q) * LANES, LANES)] = res
            return 0

        lax.fori_loop(0, CB // LANES, group_body, 0)

    fire_chunk(0, 0)
    for g in range(NCHUNK):
        buf = g % 2
        if g + 1 < NCHUNK:
            fire_chunk(g + 1, 1 - buf)
        wait_chunk(buf)
        compute_chunk(g, buf)

    pltpu.sync_copy(out_v, out.at[pl.ds(wid * BPW, BPW)])


def kernel(X, y, Wx, Wy):
    x2 = X.astype(jnp.int32).reshape(-1, IDXW)
    y2 = y.astype(jnp.int32).reshape(-1, IDXW)
    wxp, wyp = _tc_transpose(Wx.T, Wy.T)
    wxt = wxp.reshape(2 * HALF, D)
    wyt = wyp.reshape(2 * HALF, D)
    return _cbow_sc(x2, y2, wxt, wyt)


# speedup vs baseline: 2.7088x; 1.1426x over previous
import functools

import jax
import jax.numpy as jnp
from jax import lax
from jax.experimental import pallas as pl
from jax.experimental.pallas import tpu as pltpu
from jax.experimental.pallas import tpu_sc as plsc

NC, NS = 2, 16
NW = NC * NS
B = 16384
CTXW = 20
D = 64
LANES = 16
KD = D // LANES
BPW = B // NW
CB = 32
NCHUNK = BPW // CB
ROWS = CB * CTXW
IDXW = 128
NG = ROWS // IDXW
TG = BPW // IDXW

_mesh = plsc.VectorSubcoreMesh(core_axis_name="c", subcore_axis_name="s")

VOCAB = 1000000
HALF = 512000
TBN = 6400
TNB = HALF // TBN
_VLAST = (VOCAB - 1) // TBN


_TDN = (((0,), (0,)), ((), ()))


def _mt(a, eye):
    return lax.dot_general(a, eye, _TDN, preferred_element_type=jnp.float32)


def _tr_body(ax_ref, bx_ref, ay_ref, by_ref, ox_ref, oy_ref):
    p1 = jnp.eye(D, 2 * D, dtype=jnp.float32)
    p2 = jnp.eye(D, 2 * D, D, dtype=jnp.float32)
    ox_ref[...] = _mt(ax_ref[...], p1) + _mt(bx_ref[...], p2)
    oy_ref[...] = _mt(ay_ref[...], p1) + _mt(by_ref[...], p2)


def _tc_transpose(wxt, wyt):
    spec1 = pl.BlockSpec((D, TBN), lambda i: (0, i))
    spec2 = pl.BlockSpec((D, TBN), lambda i: (0, jnp.minimum(i + TNB, _VLAST)))
    ospec = pl.BlockSpec((TBN, 2 * D), lambda i: (i, 0))
    oshape = jax.ShapeDtypeStruct((HALF, 2 * D), jnp.float32)
    return pl.pallas_call(
        _tr_body,
        grid=(TNB,),
        in_specs=[spec1, spec2, spec1, spec2],
        out_specs=[ospec, ospec],
        out_shape=[oshape, oshape],
    )(wxt, wxt, wyt, wyt)


def _remap(v):
    return jnp.where(v >= HALF, v * 2 - (2 * HALF - 1), v * 2)


_GATHER_DNUMS = lax.GatherDimensionNumbers(
    offset_dims=(), collapsed_slice_dims=(0,), start_index_map=(0,))


def _dyn_gather(v, idx):
    return lax.gather(v, idx[:, None], _GATHER_DNUMS, slice_sizes=(1,),
                      mode=lax.GatherScatterMode.PROMISE_IN_BOUNDS)


@functools.partial(
    pl.kernel,
    mesh=_mesh,
    compiler_params=pltpu.CompilerParams(use_tc_tiling_on_sc=False),
    out_type=jax.ShapeDtypeStruct((B,), jnp.float32),
    scratch_types=[
        pltpu.VMEM((NCHUNK * NG, IDXW), jnp.int32),
        pltpu.VMEM((2 * TG, IDXW), jnp.int32),
        pltpu.VMEM((2, ROWS, D), jnp.float32),
        pltpu.VMEM((BPW, D), jnp.float32),
        pltpu.VMEM((BPW,), jnp.float32),
        pltpu.SemaphoreType.DMA,
        pltpu.SemaphoreType.DMA,
    ],
)
def _cbow_sc(x2, y2, wx, wy, out, cidx, tidx, ctx_v, tgt_v, out_v, sem, tsem):
    wid = lax.axis_index("s") * NC + lax.axis_index("c")

    pltpu.sync_copy(x2.at[pl.ds(wid * (NCHUNK * NG), NCHUNK * NG)], cidx)
    pltpu.sync_copy(y2.at[pl.ds((wid // 2) * (2 * TG), 2 * TG)], tidx)
    troff = (wid % 2) * TG

    def remap_row(r, _):
        for k in range(IDXW // LANES):
            sl = pl.ds(k * LANES, LANES)
            cidx[r, sl] = _remap(cidx[r, sl])
        return 0

    lax.fori_loop(0, NCHUNK * NG, remap_row, 0)
    for j in range(TG):
        for k in range(IDXW // LANES):
            sl = pl.ds(k * LANES, LANES)
            tidx[troff + j, sl] = _remap(tidx[troff + j, sl])

    for j in range(TG):
        pltpu.make_async_copy(wy.at[tidx.at[troff + j]],
                              tgt_v.at[pl.ds(j * IDXW, IDXW)], tsem).start()
    pltpu.make_async_copy(wy.at[pl.ds(0, BPW)], tgt_v, tsem).wait()

    lane = lax.broadcasted_iota(jnp.int32, (LANES,), 0)

    def fire_chunk(g, buf):
        for j in range(NG):
            pltpu.make_async_copy(
                wx.at[cidx.at[g * NG + j]],
                ctx_v.at[buf, pl.ds(j * IDXW, IDXW)], sem).start()

    def wait_chunk(buf):
        pltpu.make_async_copy(wx.at[pl.ds(0, ROWS)], ctx_v.at[buf], sem).wait()

    def compute_chunk(g, buf):
        cbuf = ctx_v.at[buf]

        def group_body(q, _):
            def batch_body(i, res):
                b = q * LANES + i
                r0 = b * CTXW
                accs = [cbuf[r0, pl.ds(k * LANES, LANES)] for k in range(KD)]
                for c in range(1, CTXW):
                    for k in range(KD):
                        accs[k] = accs[k] + cbuf[r0 + c, pl.ds(k * LANES, LANES)]
                tb = g * CB + b
                s = None
                for k in range(KD):
                    term = accs[k] * tgt_v[tb, pl.ds(k * LANES, LANES)]
                    s = term if s is None else s + term
                s = s * (1.0 / CTXW)
                for off in (8, 4, 2, 1):
                    s = s + _dyn_gather(s, lane ^ off)
                return jnp.where(lane == i, s, res)

            res = lax.fori_loop(0, LANES, batch_body,
                                jnp.zeros((LANES,), jnp.float32))
            out_v[pl.ds((g * (CB // LANES) + q) * LANES, LANES)] = res
            return 0

        lax.fori_loop(0, CB // LANES, group_body, 0)

    fire_chunk(0, 0)
    for g in range(NCHUNK):
        buf = g % 2
        if g + 1 < NCHUNK:
            fire_chunk(g + 1, 1 - buf)
        wait_chunk(buf)
        compute_chunk(g, buf)

    pltpu.sync_copy(out_v, out.at[pl.ds(wid * BPW, BPW)])


def kernel(X, y, Wx, Wy):
    x2 = X.astype(jnp.int32).reshape(-1, IDXW)
    y2 = y.astype(jnp.int32).reshape(-1, IDXW)
    wxp, wyp = _tc_transpose(Wx.T, Wy.T)
    wxt = wxp.reshape(2 * HALF, D)
    wyt = wyp.reshape(2 * HALF, D)
    return _cbow_sc(x2, y2, wxt, wyt)


# speedup vs baseline: 2.8309x; 1.0451x over previous
import functools

import jax
import jax.numpy as jnp
from jax import lax
from jax.experimental import pallas as pl
from jax.experimental.pallas import tpu as pltpu
from jax.experimental.pallas import tpu_sc as plsc

NC, NS = 2, 16
NW = NC * NS
B = 16384
CTXW = 20
D = 64
LANES = 16
KD = D // LANES
BPW = B // NW
CB = 32
NCHUNK = BPW // CB
ROWS = CB * CTXW
IDXW = 128
NG = ROWS // IDXW
TG = BPW // IDXW

_mesh = plsc.VectorSubcoreMesh(core_axis_name="c", subcore_axis_name="s")

VOCAB = 1000000
HALF = 512000
TBN = 10240
TNB = HALF // TBN
_VLAST = (VOCAB - 1) // TBN


_TDN = (((0,), (0,)), ((), ()))


def _mt(a, eye):
    return lax.dot_general(a, eye, _TDN, preferred_element_type=jnp.float32)


def _tr_body(ax_ref, bx_ref, ay_ref, by_ref, ox_ref, oy_ref):
    p1 = jnp.eye(D, 2 * D, dtype=jnp.float32)
    p2 = jnp.eye(D, 2 * D, D, dtype=jnp.float32)
    ox_ref[...] = _mt(ax_ref[...], p1) + _mt(bx_ref[...], p2)
    oy_ref[...] = _mt(ay_ref[...], p1) + _mt(by_ref[...], p2)


def _tc_transpose(wxt, wyt):
    spec1 = pl.BlockSpec((D, TBN), lambda i: (0, i))
    spec2 = pl.BlockSpec((D, TBN), lambda i: (0, jnp.minimum(i + TNB, _VLAST)))
    ospec = pl.BlockSpec((TBN, 2 * D), lambda i: (i, 0))
    oshape = jax.ShapeDtypeStruct((HALF, 2 * D), jnp.float32)
    return pl.pallas_call(
        _tr_body,
        grid=(TNB,),
        in_specs=[spec1, spec2, spec1, spec2],
        out_specs=[ospec, ospec],
        out_shape=[oshape, oshape],
    )(wxt, wxt, wyt, wyt)


def _remap(v):
    return jnp.where(v >= HALF, v * 2 - (2 * HALF - 1), v * 2)


_GATHER_DNUMS = lax.GatherDimensionNumbers(
    offset_dims=(), collapsed_slice_dims=(0,), start_index_map=(0,))


def _dyn_gather(v, idx):
    return lax.gather(v, idx[:, None], _GATHER_DNUMS, slice_sizes=(1,),
                      mode=lax.GatherScatterMode.PROMISE_IN_BOUNDS)


@functools.partial(
    pl.kernel,
    mesh=_mesh,
    compiler_params=pltpu.CompilerParams(use_tc_tiling_on_sc=False),
    out_type=jax.ShapeDtypeStruct((B,), jnp.float32),
    scratch_types=[
        pltpu.VMEM((NCHUNK * NG, IDXW), jnp.int32),
        pltpu.VMEM((2 * TG, IDXW), jnp.int32),
        pltpu.VMEM((2, ROWS, D), jnp.float32),
        pltpu.VMEM((BPW, D), jnp.float32),
        pltpu.VMEM((BPW,), jnp.float32),
        pltpu.SemaphoreType.DMA,
        pltpu.SemaphoreType.DMA,
    ],
)
def _cbow_sc(x2, y2, wx, wy, out, cidx, tidx, ctx_v, tgt_v, out_v, sem, tsem):
    wid = lax.axis_index("s") * NC + lax.axis_index("c")

    pltpu.sync_copy(x2.at[pl.ds(wid * (NCHUNK * NG), NCHUNK * NG)], cidx)
    pltpu.sync_copy(y2.at[pl.ds((wid // 2) * (2 * TG), 2 * TG)], tidx)
    troff = (wid % 2) * TG

    def remap_row(r, _):
        for k in range(IDXW // LANES):
            sl = pl.ds(k * LANES, LANES)
            cidx[r, sl] = _remap(cidx[r, sl])
        return 0

    lax.fori_loop(0, NCHUNK * NG, remap_row, 0)
    for j in range(TG):
        for k in range(IDXW // LANES):
            sl = pl.ds(k * LANES, LANES)
            tidx[troff + j, sl] = _remap(tidx[troff + j, sl])

    for j in range(TG):
        pltpu.make_async_copy(wy.at[tidx.at[troff + j]],
                              tgt_v.at[pl.ds(j * IDXW, IDXW)], tsem).start()
    pltpu.make_async_copy(wy.at[pl.ds(0, BPW)], tgt_v, tsem).wait()

    lane = lax.broadcasted_iota(jnp.int32, (LANES,), 0)

    def fire_chunk(g, buf):
        for j in range(NG):
            pltpu.make_async_copy(
                wx.at[cidx.at[g * NG + j]],
                ctx_v.at[buf, pl.ds(j * IDXW, IDXW)], sem).start()

    def wait_chunk(buf):
        pltpu.make_async_copy(wx.at[pl.ds(0, ROWS)], ctx_v.at[buf], sem).wait()

    def compute_chunk(g, buf):
        cbuf = ctx_v.at[buf]

        def group_body(q, _):
            def batch_body(i, res):
                b = q * LANES + i
                r0 = b * CTXW
                accs = [cbuf[r0, pl.ds(k * LANES, LANES)] for k in range(KD)]
                for c in range(1, CTXW):
                    for k in range(KD):
                        accs[k] = accs[k] + cbuf[r0 + c, pl.ds(k * LANES, LANES)]
                tb = g * CB + b
                s = None
                for k in range(KD):
                    term = accs[k] * tgt_v[tb, pl.ds(k * LANES, LANES)]
                    s = term if s is None else s + term
                s = s * (1.0 / CTXW)
                for off in (8, 4, 2, 1):
                    s = s + _dyn_gather(s, lane ^ off)
                return jnp.where(lane == i, s, res)

            res = lax.fori_loop(0, LANES, batch_body,
                                jnp.zeros((LANES,), jnp.float32))
            out_v[pl.ds((g * (CB // LANES) + q) * LANES, LANES)] = res
            return 0

        lax.fori_loop(0, CB // LANES, group_body, 0)

    fire_chunk(0, 0)
    for g in range(NCHUNK):
        buf = g % 2
        if g + 1 < NCHUNK:
            fire_chunk(g + 1, 1 - buf)
        wait_chunk(buf)
        compute_chunk(g, buf)

    pltpu.sync_copy(out_v, out.at[pl.ds(wid * BPW, BPW)])


def kernel(X, y, Wx, Wy):
    x2 = X.astype(jnp.int32).reshape(-1, IDXW)
    y2 = y.astype(jnp.int32).reshape(-1, IDXW)
    wxp, wyp = _tc_transpose(Wx.T, Wy.T)
    wxt = wxp.reshape(2 * HALF, D)
    wyt = wyp.reshape(2 * HALF, D)
    return _cbow_sc(x2, y2, wxt, wyt)
